# baseline (speedup 1.0000x reference)
import functools

import jax
import jax.numpy as jnp
from jax import lax
from jax.experimental import pallas as pl
from jax.experimental.pallas import tpu as pltpu
from jax.experimental.pallas import tpu_sc as plsc

_V = 100000
_N = 4
_E = 512
_B = 4096

_info = plsc.get_sparse_core_info()
_NW = _info.num_cores * _info.num_subcores
_BPW = _B // _NW
_CP = 16
_CG = 8
_GPP = _CP // _CG
_NP = _BPW // _CP
_NBUF = 3

_mesh = plsc.VectorSubcoreMesh(core_axis_name="c", subcore_axis_name="s")


@functools.partial(
    pl.kernel,
    mesh=_mesh,
    out_type=jax.ShapeDtypeStruct((_B, _N, _E), jnp.float32),
    scratch_types=[
        pltpu.VMEM((_BPW,), jnp.int32),
        pltpu.VMEM((_NBUF, _CP, _N, _E), jnp.float32),
        pltpu.SemaphoreType.DMA((_NBUF,)),
        pltpu.SemaphoreType.DMA((_NBUF,)),
    ],
)
def _gather_kernel(pids_hbm, ctx_hbm, out_hbm, idx_v, bufs, gsems, wsems):
    wid = lax.axis_index("s") * _info.num_cores + lax.axis_index("c")
    base = wid * _BPW

    pltpu.sync_copy(pids_hbm.at[pl.ds(base, _BPW)], idx_v)

    gathers = [None] * _NP
    writes = [None] * _NP

    def start_pair(p):
        b = p % _NBUF
        gathers[p] = [
            pltpu.async_copy(
                ctx_hbm.at[idx_v.at[pl.ds(p * _CP + g * _CG, _CG)]],
                bufs.at[b, pl.ds(g * _CG, _CG)],
                gsems.at[b],
            )
            for g in range(_GPP)
        ]

    for p in range(min(_NBUF - 1, _NP)):
        start_pair(p)
    for p in range(_NP):
        if p + _NBUF - 1 < _NP:
            if p >= 1:
                writes[p - 1].wait()
            start_pair(p + _NBUF - 1)
        for g in gathers[p]:
            g.wait()
        writes[p] = pltpu.async_copy(
            bufs.at[p % _NBUF],
            out_hbm.at[pl.ds(base + p * _CP, _CP)],
            wsems.at[p % _NBUF],
        )
    for p in range(max(0, _NP - _NBUF + 1), _NP):
        writes[p].wait()


def kernel(pids, ctx):
    return _gather_kernel(pids.astype(jnp.int32), ctx)


# speedup vs baseline: 1.0353x; 1.0353x over previous
import functools

import jax
import jax.numpy as jnp
from jax import lax
from jax.experimental import pallas as pl
from jax.experimental.pallas import tpu as pltpu
from jax.experimental.pallas import tpu_sc as plsc

_V = 100000
_N = 4
_E = 512
_B = 4096

_info = plsc.get_sparse_core_info()
_NW = _info.num_cores * _info.num_subcores
_BPW = _B // _NW
_C = 8
_NCH = _BPW // _C
_NBUF = 6

_mesh = plsc.VectorSubcoreMesh(core_axis_name="c", subcore_axis_name="s")


@functools.partial(
    pl.kernel,
    mesh=_mesh,
    out_type=jax.ShapeDtypeStruct((_B, _N, _E), jnp.float32),
    scratch_types=[
        pltpu.VMEM((_BPW,), jnp.int32),
        pltpu.VMEM((_NBUF, _C, _N, _E), jnp.float32),
        pltpu.SemaphoreType.DMA((_NBUF,)),
        pltpu.SemaphoreType.DMA((_NBUF,)),
    ],
)
def _gather_kernel(pids_hbm, ctx_hbm, out_hbm, idx_v, bufs, gsems, wsems):
    wid = lax.axis_index("c") * _info.num_subcores + lax.axis_index("s")
    base = wid * _BPW

    pltpu.sync_copy(pids_hbm.at[pl.ds(base, _BPW)], idx_v)

    gathers = [None] * _NCH
    writes = [None] * _NCH

    def start_gather(c):
        gathers[c] = pltpu.async_copy(
            ctx_hbm.at[idx_v.at[pl.ds(c * _C, _C)]],
            bufs.at[c % _NBUF],
            gsems.at[c % _NBUF],
        )

    for c in range(min(_NBUF - 1, _NCH)):
        start_gather(c)
    for c in range(_NCH):
        if c + _NBUF - 1 < _NCH:
            if c >= 1:
                writes[c - 1].wait()
            start_gather(c + _NBUF - 1)
        gathers[c].wait()
        writes[c] = pltpu.async_copy(
            bufs.at[c % _NBUF],
            out_hbm.at[pl.ds(base + c * _C, _C)],
            wsems.at[c % _NBUF],
        )
    for c in range(max(0, _NCH - _NBUF + 1), _NCH):
        writes[c].wait()


def kernel(pids, ctx):
    return _gather_kernel(pids.astype(jnp.int32), ctx)


# speedup vs baseline: 1.0364x; 1.0011x over previous
import functools

import jax
import jax.numpy as jnp
from jax import lax
from jax.experimental import pallas as pl
from jax.experimental.pallas import tpu as pltpu
from jax.experimental.pallas import tpu_sc as plsc

_V = 100000
_N = 4
_E = 512
_B = 4096

_info = plsc.get_sparse_core_info()
_NW = _info.num_cores * _info.num_subcores
_BPW = _B // _NW
_C = 8
_NCH = _BPW // _C
_NBUF = 6

_mesh = plsc.VectorSubcoreMesh(core_axis_name="c", subcore_axis_name="s")


@functools.partial(
    pl.kernel,
    mesh=_mesh,
    out_type=jax.ShapeDtypeStruct((_B, _N, _E), jnp.float32),
    scratch_types=[
        pltpu.VMEM((_BPW,), jnp.int32),
        pltpu.VMEM((_NBUF, _C, _N, _E), jnp.float32),
        pltpu.SemaphoreType.DMA((_NBUF,)),
        pltpu.SemaphoreType.DMA((_NBUF,)),
    ],
)
def _gather_kernel(pids_hbm, ctx_hbm, out_hbm, idx_v, bufs, gsems, wsems):
    wid = lax.axis_index("c") * _info.num_subcores + lax.axis_index("s")
    base = wid * _BPW

    pltpu.sync_copy(pids_hbm.at[pl.ds(base, _BPW)], idx_v)

    def gather_copy(c):
        b = lax.rem(c, _NBUF) if not isinstance(c, int) else c % _NBUF
        return pltpu.make_async_copy(
            ctx_hbm.at[idx_v.at[pl.ds(pl.multiple_of(c * _C, 8), _C)]],
            bufs.at[b],
            gsems.at[b],
        )

    def write_copy(c):
        b = lax.rem(c, _NBUF) if not isinstance(c, int) else c % _NBUF
        return pltpu.make_async_copy(
            bufs.at[b],
            out_hbm.at[pl.ds(pl.multiple_of(base + c * _C, 8), _C)],
            wsems.at[b],
        )

    for c in range(_NBUF - 1):
        gather_copy(c).start()

    def body(c, carry):
        @pl.when(c + _NBUF - 1 < _NCH)
        def _():
            @pl.when(c >= 1)
            def _():
                write_copy(c - 1).wait()
            gather_copy(c + _NBUF - 1).start()
        gather_copy(c).wait()
        write_copy(c).start()
        return carry

    lax.fori_loop(0, _NCH, body, 0)
    for c in range(_NCH - _NBUF + 1, _NCH):
        write_copy(c).wait()


def kernel(pids, ctx):
    return _gather_kernel(pids.astype(jnp.int32), ctx)
